# Initial kernel scaffold; baseline (speedup 1.0000x reference)
import jax
import jax.numpy as jnp
from jax.experimental import pallas as pl


def kernel(x0, x1, P0_W, P0_b, P1_W, P1_b, bases0, comp0, root0, bias0, bases1, comp1, root1, bias1, edge_index, edge_type):
    raise NotImplementedError("write your pallas kernel here")


# speedup vs baseline: 15.6898x; 15.6898x over previous
import functools

import jax
import jax.numpy as jnp
from jax import lax
from jax.experimental import pallas as pl
from jax.experimental.pallas import tpu as pltpu
from jax.experimental.pallas import tpu_sc as plsc

N0, N1 = 6000, 4000
N = N0 + N1
E = 320000
D = 128
R = 3

NC, NS = 2, 16
NW = NC * NS
EPT = E // NW
CHUNK = 80
NCHUNK = EPT // CHUNK
ZR = 80
NZB = N // ZR
BPT = (NZB + NS - 1) // NS
DEGW = 128

_F32 = jnp.float32
_I32 = jnp.int32


@functools.lru_cache(maxsize=None)
def _make_sc_agg():
  mesh = plsc.VectorSubcoreMesh(core_axis_name="c", subcore_axis_name="s")
  out_type = [jax.ShapeDtypeStruct((NC * N, D), _F32)]
  scratch = [
      pltpu.VMEM((CHUNK,), _I32),
      pltpu.VMEM((CHUNK,), _I32),
      pltpu.VMEM((ZR,), _I32),
      pltpu.VMEM((CHUNK, D), _F32),
      pltpu.VMEM((ZR, D), _F32),
      pltpu.VMEM_SHARED((N, D), _F32),
      pltpu.SemaphoreType.DMA,
  ]

  def body(hall, gidx_h, dst_h, rowidx_h, zrow_h,
           agg_out, gidx_v, dst_v, ridx_v, rows_v, zrow_v, acc, sem):
    c = lax.axis_index("c")
    s = lax.axis_index("s")
    wid = c * NS + s
    base = wid * EPT

    pltpu.sync_copy(zrow_h, zrow_v)

    for k in range(BPT):
      blk = jnp.minimum(s * BPT + k, NZB - 1)
      pltpu.sync_copy(rowidx_h.at[pl.ds(blk * ZR, ZR)], ridx_v)
      pltpu.sync_copy(zrow_v, acc.at[ridx_v])
    plsc.subcore_barrier()

    def chunk_body(i, _):
      off = base + i * CHUNK
      pltpu.sync_copy(gidx_h.at[pl.ds(off, CHUNK)], gidx_v)
      pltpu.sync_copy(dst_h.at[pl.ds(off, CHUNK)], dst_v)
      pltpu.async_copy(hall.at[gidx_v], rows_v, sem).wait()
      pltpu.sync_copy(rows_v, acc.at[dst_v], add=True)
      return 0
    lax.fori_loop(0, NCHUNK, chunk_body, 0)

    plsc.subcore_barrier()
    for k in range(BPT):
      blk = jnp.minimum(s * BPT + k, NZB - 1)
      pltpu.sync_copy(rowidx_h.at[pl.ds(blk * ZR, ZR)], ridx_v)
      pltpu.async_copy(acc.at[ridx_v], rows_v, sem).wait()
      pltpu.sync_copy(rows_v, agg_out.at[pl.ds(c * N + blk * ZR, ZR)])

  return pl.kernel(body, out_type=out_type, mesh=mesh,
                   scratch_types=scratch)


@functools.lru_cache(maxsize=None)
def _make_sc_deg():
  mesh = plsc.VectorSubcoreMesh(core_axis_name="c", subcore_axis_name="s")
  out_type = [jax.ShapeDtypeStruct((NC * N, DEGW), _F32)]
  scratch = [
      pltpu.VMEM((CHUNK,), _I32),
      pltpu.VMEM((ZR,), _I32),
      pltpu.VMEM((CHUNK, DEGW), _F32),
      pltpu.VMEM((ZR, DEGW), _F32),
      pltpu.VMEM_SHARED((N, DEGW), _F32),
      pltpu.SemaphoreType.DMA,
  ]

  def body(dst_h, rowidx_h, ones_h, zd_h,
           deg_out, dst_v, ridx_v, ones_v, zd_v, dacc, sem):
    c = lax.axis_index("c")
    s = lax.axis_index("s")
    wid = c * NS + s
    base = wid * EPT

    pltpu.sync_copy(ones_h, ones_v)
    pltpu.sync_copy(zd_h, zd_v)

    for k in range(BPT):
      blk = jnp.minimum(s * BPT + k, NZB - 1)
      pltpu.sync_copy(rowidx_h.at[pl.ds(blk * ZR, ZR)], ridx_v)
      pltpu.sync_copy(zd_v, dacc.at[ridx_v])
    plsc.subcore_barrier()

    def chunk_body(i, _):
      off = base + i * CHUNK
      pltpu.sync_copy(dst_h.at[pl.ds(off, CHUNK)], dst_v)
      pltpu.sync_copy(ones_v, dacc.at[dst_v], add=True)
      return 0
    lax.fori_loop(0, NCHUNK, chunk_body, 0)

    plsc.subcore_barrier()
    for k in range(BPT):
      blk = jnp.minimum(s * BPT + k, NZB - 1)
      pltpu.sync_copy(rowidx_h.at[pl.ds(blk * ZR, ZR)], ridx_v)
      pltpu.async_copy(dacc.at[ridx_v], zd_v, sem).wait()
      pltpu.sync_copy(zd_v, deg_out.at[pl.ds(c * N + blk * ZR, ZR)])

  return pl.kernel(body, out_type=out_type, mesh=mesh,
                   scratch_types=scratch)


def _dot(a, b):
  return jnp.dot(a, b, preferred_element_type=_F32)


def _tc_pre_body(x0, x1, p0w, p0b, p1w, p1b, bases, comp, root, src2d, typ2d,
                 lat0, hall, xr, gidx2d):
  h0 = jnp.maximum(_dot(x0[...], p0w[...]) + p0b[...][None, :], 0.0)
  h1 = jnp.maximum(_dot(x1[...], p1w[...]) + p1b[...][None, :], 0.0)
  lat0[0:N0, :] = h0
  lat0[N0:N, :] = h1
  x = jnp.concatenate([h0, h1], axis=0)
  cm = comp[...]
  b = bases[...]
  for r in range(R):
    w_r = cm[r, 0] * b[0] + cm[r, 1] * b[1]
    hall[r * N:(r + 1) * N, :] = _dot(x, w_r)
  xr[...] = _dot(x, root[...])
  gidx2d[...] = typ2d[...] * N + src2d[...]


_tc_pre = pl.pallas_call(
    _tc_pre_body,
    out_shape=[
        jax.ShapeDtypeStruct((N, D), _F32),
        jax.ShapeDtypeStruct((R * N, D), _F32),
        jax.ShapeDtypeStruct((N, D), _F32),
        jax.ShapeDtypeStruct((E // D, D), _I32),
    ],
)


def _tc_mid_body(aggp, degp, xr0, bias0, bases, comp, root,
                 lat1, hall, xr1):
  a = aggp[...]
  dg = degp[...]
  deg = jnp.maximum(dg[0, :, 0] + dg[1, :, 0], 1.0)
  y = (a[0] + a[1]) / deg[:, None] + xr0[...] + bias0[...][None, :]
  y = jnp.maximum(y, 0.0)
  y = y - jnp.mean(y, axis=0, keepdims=True)
  y = y * lax.rsqrt(1e-5 + jnp.sum(y * y) / N)
  lat1[...] = y
  cm = comp[...]
  b = bases[...]
  for r in range(R):
    w_r = cm[r, 0] * b[0] + cm[r, 1] * b[1]
    hall[r * N:(r + 1) * N, :] = _dot(y, w_r)
  xr1[...] = _dot(y, root[...])


_tc_mid = pl.pallas_call(
    _tc_mid_body,
    out_shape=[
        jax.ShapeDtypeStruct((N, D), _F32),
        jax.ShapeDtypeStruct((R * N, D), _F32),
        jax.ShapeDtypeStruct((N, D), _F32),
    ],
)


def _tc_post_body(aggp, degp, xr1, bias1, out):
  a = aggp[...]
  dg = degp[...]
  deg = jnp.maximum(dg[0, :, 0] + dg[1, :, 0], 1.0)
  y = (a[0] + a[1]) / deg[:, None] + xr1[...] + bias1[...][None, :]
  y = y - jnp.mean(y, axis=0, keepdims=True)
  out[...] = y * lax.rsqrt(1e-5 + jnp.sum(y * y) / N)


_tc_post = pl.pallas_call(
    _tc_post_body,
    out_shape=jax.ShapeDtypeStruct((N, D), _F32),
)


def kernel(x0, x1, P0_W, P0_b, P1_W, P1_b, bases0, comp0, root0, bias0,
           bases1, comp1, root1, bias1, edge_index, edge_type):
  src2d = edge_index[0].astype(_I32).reshape(E // D, D)
  typ2d = edge_type.astype(_I32).reshape(E // D, D)
  dst = edge_index[1].astype(_I32)
  rowidx = jnp.arange(N, dtype=_I32)
  zrow = jnp.zeros((ZR, D), _F32)
  ones = jnp.ones((CHUNK, DEGW), _F32)
  zd = jnp.zeros((ZR, DEGW), _F32)

  lat0, hall0, xr0, gidx2d = _tc_pre(x0, x1, P0_W, P0_b, P1_W, P1_b,
                                     bases0, comp0, root0, src2d, typ2d)
  gidx = gidx2d.reshape(E)
  (degp,) = _make_sc_deg()(dst, rowidx, ones, zd)
  degp = degp.reshape(NC, N, DEGW)
  (aggp0,) = _make_sc_agg()(hall0, gidx, dst, rowidx, zrow)
  aggp0 = aggp0.reshape(NC, N, D)
  lat1, hall1, xr1 = _tc_mid(aggp0, degp, xr0, bias0, bases1, comp1, root1)
  (aggp1,) = _make_sc_agg()(hall1, gidx, dst, rowidx, zrow)
  aggp1 = aggp1.reshape(NC, N, D)
  out = _tc_post(aggp1, degp, xr1, bias1)
  return (out, lat0, lat1)


# speedup vs baseline: 22.4882x; 1.4333x over previous
import functools

import jax
import jax.numpy as jnp
from jax import lax
from jax.experimental import pallas as pl
from jax.experimental.pallas import tpu as pltpu
from jax.experimental.pallas import tpu_sc as plsc

N0, N1 = 6000, 4000
N = N0 + N1
E = 320000
D = 128
R = 3

NC, NS = 2, 16
NW = NC * NS
EPT = E // NW
CHUNK = 80
NCHUNK = EPT // CHUNK
ZR = 80
NZB = N // ZR
BPT = (NZB + NS - 1) // NS
DEGW = 128

_F32 = jnp.float32
_I32 = jnp.int32


@functools.lru_cache(maxsize=None)
def _make_sc_agg():
  mesh = plsc.VectorSubcoreMesh(core_axis_name="c", subcore_axis_name="s")
  out_type = [jax.ShapeDtypeStruct((NC * N, D), _F32)]
  scratch = [
      pltpu.VMEM((CHUNK,), _I32),
      pltpu.VMEM((CHUNK,), _I32),
      pltpu.VMEM((CHUNK,), _I32),
      pltpu.VMEM((CHUNK,), _I32),
      pltpu.VMEM((ZR,), _I32),
      pltpu.VMEM((CHUNK, D), _F32),
      pltpu.VMEM((CHUNK, D), _F32),
      pltpu.VMEM((ZR, D), _F32),
      pltpu.VMEM_SHARED((N, D), _F32),
      pltpu.SemaphoreType.DMA,
      pltpu.SemaphoreType.DMA,
  ]

  def body(hall, gidx_h, dst_h, rowidx_h, zrow_h,
           agg_out, gidx_v0, dst_v0, gidx_v1, dst_v1, ridx_v,
           rows_v0, rows_v1, zrow_v, acc, sem0, sem1):
    c = lax.axis_index("c")
    s = lax.axis_index("s")
    wid = c * NS + s
    base = wid * EPT

    pltpu.sync_copy(zrow_h, zrow_v)

    for k in range(BPT):
      blk = jnp.minimum(s * BPT + k, NZB - 1)
      pltpu.sync_copy(rowidx_h.at[pl.ds(blk * ZR, ZR)], ridx_v)
      pltpu.sync_copy(zrow_v, acc.at[ridx_v])
    plsc.subcore_barrier()

    bufs = ((gidx_v0, dst_v0, rows_v0, sem0),
            (gidx_v1, dst_v1, rows_v1, sem1))

    def start(i, b):
      gv, dv, rv, sm = bufs[b]
      off = base + i * CHUNK
      pltpu.sync_copy(gidx_h.at[pl.ds(off, CHUNK)], gv)
      pltpu.sync_copy(dst_h.at[pl.ds(off, CHUNK)], dv)
      pltpu.async_copy(hall.at[gv], rv, sm)

    def finish(b):
      gv, dv, rv, sm = bufs[b]
      pltpu.make_async_copy(hall.at[gv], rv, sm).wait()
      pltpu.sync_copy(rv, acc.at[dv], add=True)

    start(0, 0)

    def chunk_pair(t, _):
      g = 2 * t
      start(g + 1, 1)
      finish(0)
      start(g + 2, 0)
      finish(1)
      return 0
    lax.fori_loop(0, (NCHUNK - 1) // 2, chunk_pair, 0)
    finish(0)

    plsc.subcore_barrier()
    for k in range(BPT):
      blk = jnp.minimum(s * BPT + k, NZB - 1)
      pltpu.sync_copy(rowidx_h.at[pl.ds(blk * ZR, ZR)], ridx_v)
      pltpu.async_copy(acc.at[ridx_v], rows_v0, sem0).wait()
      pltpu.sync_copy(rows_v0, agg_out.at[pl.ds(c * N + blk * ZR, ZR)])

  return pl.kernel(body, out_type=out_type, mesh=mesh,
                   scratch_types=scratch)


@functools.lru_cache(maxsize=None)
def _make_sc_deg():
  mesh = plsc.VectorSubcoreMesh(core_axis_name="c", subcore_axis_name="s")
  out_type = [jax.ShapeDtypeStruct((NC * N, DEGW), _F32)]
  scratch = [
      pltpu.VMEM((CHUNK,), _I32),
      pltpu.VMEM((ZR,), _I32),
      pltpu.VMEM((CHUNK, DEGW), _F32),
      pltpu.VMEM((ZR, DEGW), _F32),
      pltpu.VMEM_SHARED((N, DEGW), _F32),
      pltpu.SemaphoreType.DMA,
  ]

  def body(dst_h, rowidx_h, ones_h, zd_h,
           deg_out, dst_v, ridx_v, ones_v, zd_v, dacc, sem):
    c = lax.axis_index("c")
    s = lax.axis_index("s")
    wid = c * NS + s
    base = wid * EPT

    pltpu.sync_copy(ones_h, ones_v)
    pltpu.sync_copy(zd_h, zd_v)

    for k in range(BPT):
      blk = jnp.minimum(s * BPT + k, NZB - 1)
      pltpu.sync_copy(rowidx_h.at[pl.ds(blk * ZR, ZR)], ridx_v)
      pltpu.sync_copy(zd_v, dacc.at[ridx_v])
    plsc.subcore_barrier()

    def chunk_body(i, _):
      off = base + i * CHUNK
      pltpu.sync_copy(dst_h.at[pl.ds(off, CHUNK)], dst_v)
      pltpu.sync_copy(ones_v, dacc.at[dst_v], add=True)
      return 0
    lax.fori_loop(0, NCHUNK, chunk_body, 0)

    plsc.subcore_barrier()
    for k in range(BPT):
      blk = jnp.minimum(s * BPT + k, NZB - 1)
      pltpu.sync_copy(rowidx_h.at[pl.ds(blk * ZR, ZR)], ridx_v)
      pltpu.async_copy(dacc.at[ridx_v], zd_v, sem).wait()
      pltpu.sync_copy(zd_v, deg_out.at[pl.ds(c * N + blk * ZR, ZR)])

  return pl.kernel(body, out_type=out_type, mesh=mesh,
                   scratch_types=scratch)


def _dot(a, b):
  return jnp.dot(a, b, preferred_element_type=_F32)


def _tc_pre_body(x0, x1, p0w, p0b, p1w, p1b, bases, comp, root, src2d, typ2d,
                 lat0, hall, xr, gidx2d):
  h0 = jnp.maximum(_dot(x0[...], p0w[...]) + p0b[...][None, :], 0.0)
  h1 = jnp.maximum(_dot(x1[...], p1w[...]) + p1b[...][None, :], 0.0)
  lat0[0:N0, :] = h0
  lat0[N0:N, :] = h1
  x = jnp.concatenate([h0, h1], axis=0)
  cm = comp[...]
  b = bases[...]
  for r in range(R):
    w_r = cm[r, 0] * b[0] + cm[r, 1] * b[1]
    hall[r * N:(r + 1) * N, :] = _dot(x, w_r)
  xr[...] = _dot(x, root[...])
  gidx2d[...] = typ2d[...] * N + src2d[...]


_tc_pre = pl.pallas_call(
    _tc_pre_body,
    out_shape=[
        jax.ShapeDtypeStruct((N, D), _F32),
        jax.ShapeDtypeStruct((R * N, D), _F32),
        jax.ShapeDtypeStruct((N, D), _F32),
        jax.ShapeDtypeStruct((E // D, D), _I32),
    ],
)


def _tc_mid_body(aggp, degp, xr0, bias0, bases, comp, root,
                 lat1, hall, xr1):
  a = aggp[...]
  dg = degp[...]
  deg = jnp.maximum(dg[0, :, 0] + dg[1, :, 0], 1.0)
  y = (a[0] + a[1]) / deg[:, None] + xr0[...] + bias0[...][None, :]
  y = jnp.maximum(y, 0.0)
  y = y - jnp.mean(y, axis=0, keepdims=True)
  y = y * lax.rsqrt(1e-5 + jnp.sum(y * y) / N)
  lat1[...] = y
  cm = comp[...]
  b = bases[...]
  for r in range(R):
    w_r = cm[r, 0] * b[0] + cm[r, 1] * b[1]
    hall[r * N:(r + 1) * N, :] = _dot(y, w_r)
  xr1[...] = _dot(y, root[...])


_tc_mid = pl.pallas_call(
    _tc_mid_body,
    out_shape=[
        jax.ShapeDtypeStruct((N, D), _F32),
        jax.ShapeDtypeStruct((R * N, D), _F32),
        jax.ShapeDtypeStruct((N, D), _F32),
    ],
)


def _tc_post_body(aggp, degp, xr1, bias1, out):
  a = aggp[...]
  dg = degp[...]
  deg = jnp.maximum(dg[0, :, 0] + dg[1, :, 0], 1.0)
  y = (a[0] + a[1]) / deg[:, None] + xr1[...] + bias1[...][None, :]
  y = y - jnp.mean(y, axis=0, keepdims=True)
  out[...] = y * lax.rsqrt(1e-5 + jnp.sum(y * y) / N)


_tc_post = pl.pallas_call(
    _tc_post_body,
    out_shape=jax.ShapeDtypeStruct((N, D), _F32),
)


def kernel(x0, x1, P0_W, P0_b, P1_W, P1_b, bases0, comp0, root0, bias0,
           bases1, comp1, root1, bias1, edge_index, edge_type):
  src2d = edge_index[0].astype(_I32).reshape(E // D, D)
  typ2d = edge_type.astype(_I32).reshape(E // D, D)
  dst = edge_index[1].astype(_I32)
  rowidx = jnp.arange(N, dtype=_I32)
  zrow = jnp.zeros((ZR, D), _F32)
  ones = jnp.ones((CHUNK, DEGW), _F32)
  zd = jnp.zeros((ZR, DEGW), _F32)

  lat0, hall0, xr0, gidx2d = _tc_pre(x0, x1, P0_W, P0_b, P1_W, P1_b,
                                     bases0, comp0, root0, src2d, typ2d)
  gidx = gidx2d.reshape(E)
  (degp,) = _make_sc_deg()(dst, rowidx, ones, zd)
  degp = degp.reshape(NC, N, DEGW)
  (aggp0,) = _make_sc_agg()(hall0, gidx, dst, rowidx, zrow)
  aggp0 = aggp0.reshape(NC, N, D)
  lat1, hall1, xr1 = _tc_mid(aggp0, degp, xr0, bias0, bases1, comp1, root1)
  (aggp1,) = _make_sc_agg()(hall1, gidx, dst, rowidx, zrow)
  aggp1 = aggp1.reshape(NC, N, D)
  out = _tc_post(aggp1, degp, xr1, bias1)
  return (out, lat0, lat1)


# speedup vs baseline: 25.5410x; 1.1358x over previous
import functools

import jax
import jax.numpy as jnp
from jax import lax
from jax.experimental import pallas as pl
from jax.experimental.pallas import tpu as pltpu
from jax.experimental.pallas import tpu_sc as plsc

N0, N1 = 6000, 4000
N = N0 + N1
E = 320000
D = 128
R = 3

NC, NS = 2, 16
NW = NC * NS
EPT = E // NW
CHUNK = 80
NCHUNK = EPT // CHUNK
ZR = 80
NZB = N // ZR
BPT = (NZB + NS - 1) // NS
HPAD = 10240

_F32 = jnp.float32
_I32 = jnp.int32


@functools.lru_cache(maxsize=None)
def _make_sc_agg():
  mesh = plsc.VectorSubcoreMesh(core_axis_name="c", subcore_axis_name="s")
  out_type = [jax.ShapeDtypeStruct((NC * N, D), _F32)]
  scratch = [
      pltpu.VMEM((CHUNK,), _I32),
      pltpu.VMEM((CHUNK,), _I32),
      pltpu.VMEM((CHUNK,), _I32),
      pltpu.VMEM((CHUNK,), _I32),
      pltpu.VMEM((ZR,), _I32),
      pltpu.VMEM((CHUNK, D), _F32),
      pltpu.VMEM((CHUNK, D), _F32),
      pltpu.VMEM((ZR, D), _F32),
      pltpu.VMEM_SHARED((N, D), _F32),
      pltpu.SemaphoreType.DMA,
      pltpu.SemaphoreType.DMA,
  ]

  def body(hall, gidx_h, dst_h, rowidx_h, zrow_h,
           agg_out, gidx_v0, dst_v0, gidx_v1, dst_v1, ridx_v,
           rows_v0, rows_v1, zrow_v, acc, sem0, sem1):
    c = lax.axis_index("c")
    s = lax.axis_index("s")
    wid = c * NS + s
    base = wid * EPT

    pltpu.sync_copy(zrow_h, zrow_v)

    for k in range(BPT):
      blk = jnp.minimum(s * BPT + k, NZB - 1)
      pltpu.sync_copy(rowidx_h.at[pl.ds(blk * ZR, ZR)], ridx_v)
      pltpu.sync_copy(zrow_v, acc.at[ridx_v])
    plsc.subcore_barrier()

    bufs = ((gidx_v0, dst_v0, rows_v0, sem0),
            (gidx_v1, dst_v1, rows_v1, sem1))

    def start(i, b):
      gv, dv, rv, sm = bufs[b]
      off = base + i * CHUNK
      pltpu.sync_copy(gidx_h.at[pl.ds(off, CHUNK)], gv)
      pltpu.sync_copy(dst_h.at[pl.ds(off, CHUNK)], dv)
      pltpu.async_copy(hall.at[gv], rv, sm)

    def finish(b):
      gv, dv, rv, sm = bufs[b]
      pltpu.make_async_copy(hall.at[gv], rv, sm).wait()
      pltpu.sync_copy(rv, acc.at[dv], add=True)

    start(0, 0)

    def chunk_pair(t, _):
      g = 2 * t
      start(g + 1, 1)
      finish(0)
      start(g + 2, 0)
      finish(1)
      return 0
    lax.fori_loop(0, (NCHUNK - 1) // 2, chunk_pair, 0)
    finish(0)

    plsc.subcore_barrier()
    for k in range(BPT):
      blk = jnp.minimum(s * BPT + k, NZB - 1)
      pltpu.sync_copy(rowidx_h.at[pl.ds(blk * ZR, ZR)], ridx_v)
      pltpu.async_copy(acc.at[ridx_v], rows_v0, sem0).wait()
      pltpu.sync_copy(rows_v0, agg_out.at[pl.ds(c * N + blk * ZR, ZR)])

  return pl.kernel(body, out_type=out_type, mesh=mesh,
                   scratch_types=scratch)


@functools.lru_cache(maxsize=None)
def _make_sc_deg():
  mesh = plsc.VectorSubcoreMesh(core_axis_name="c", subcore_axis_name="s")
  NH = NZB * ZR * 128 // 125
  out_type = [jax.ShapeDtypeStruct((NW * HPAD,), _F32)]
  scratch = [
      pltpu.VMEM((CHUNK,), _I32),
      pltpu.VMEM((HPAD,), _F32),
      pltpu.SemaphoreType.DMA,
  ]

  def body(dst_h, zz_h, deg_out, dst_v, hist_v, sem):
    c = lax.axis_index("c")
    s = lax.axis_index("s")
    wid = c * NS + s
    base = wid * EPT

    pltpu.sync_copy(zz_h, hist_v)
    ones = jnp.ones((16,), _F32)

    def chunk_body(i, _):
      off = base + i * CHUNK
      pltpu.sync_copy(dst_h.at[pl.ds(off, CHUNK)], dst_v)
      for j in range(CHUNK // 16):
        d = dst_v[pl.ds(j * 16, 16)]
        plsc.addupdate_scatter(hist_v, [d], ones)
      return 0
    lax.fori_loop(0, NCHUNK, chunk_body, 0)

    pltpu.sync_copy(hist_v, deg_out.at[pl.ds(wid * HPAD, HPAD)])

  return pl.kernel(body, out_type=out_type, mesh=mesh,
                   compiler_params=pltpu.CompilerParams(
                       needs_layout_passes=False),
                   scratch_types=scratch)


def _dot(a, b):
  return jnp.dot(a, b, preferred_element_type=_F32)


def _tc_pre_body(x0, x1, p0w, p0b, p1w, p1b, bases, comp, root, src2d, typ2d,
                 lat0, hall, xr, gidx2d):
  h0 = jnp.maximum(_dot(x0[...], p0w[...]) + p0b[...][None, :], 0.0)
  h1 = jnp.maximum(_dot(x1[...], p1w[...]) + p1b[...][None, :], 0.0)
  lat0[0:N0, :] = h0
  lat0[N0:N, :] = h1
  x = jnp.concatenate([h0, h1], axis=0)
  cm = comp[...]
  b = bases[...]
  for r in range(R):
    w_r = cm[r, 0] * b[0] + cm[r, 1] * b[1]
    hall[r * N:(r + 1) * N, :] = _dot(x, w_r)
  xr[...] = _dot(x, root[...])
  gidx2d[...] = typ2d[...] * N + src2d[...]


_tc_pre = pl.pallas_call(
    _tc_pre_body,
    out_shape=[
        jax.ShapeDtypeStruct((N, D), _F32),
        jax.ShapeDtypeStruct((R * N, D), _F32),
        jax.ShapeDtypeStruct((N, D), _F32),
        jax.ShapeDtypeStruct((E // D, D), _I32),
    ],
)


def _tc_degsum_body(degp, dinv):
  dinv[...] = 1.0 / jnp.maximum(jnp.sum(degp[...], axis=0), 1.0)


_tc_degsum = pl.pallas_call(
    _tc_degsum_body,
    out_shape=jax.ShapeDtypeStruct((HPAD // 128, 128), _F32),
)


def _tc_mid_body(aggp, dinv, xr0, bias0, bases, comp, root,
                 lat1, hall, xr1):
  a = aggp[...]
  y = (a[0] + a[1]) * dinv[...] + xr0[...] + bias0[...][None, :]
  y = jnp.maximum(y, 0.0)
  y = y - jnp.mean(y, axis=0, keepdims=True)
  y = y * lax.rsqrt(1e-5 + jnp.sum(y * y) / N)
  lat1[...] = y
  cm = comp[...]
  b = bases[...]
  for r in range(R):
    w_r = cm[r, 0] * b[0] + cm[r, 1] * b[1]
    hall[r * N:(r + 1) * N, :] = _dot(y, w_r)
  xr1[...] = _dot(y, root[...])


_tc_mid = pl.pallas_call(
    _tc_mid_body,
    out_shape=[
        jax.ShapeDtypeStruct((N, D), _F32),
        jax.ShapeDtypeStruct((R * N, D), _F32),
        jax.ShapeDtypeStruct((N, D), _F32),
    ],
)


def _tc_post_body(aggp, dinv, xr1, bias1, out):
  a = aggp[...]
  y = (a[0] + a[1]) * dinv[...] + xr1[...] + bias1[...][None, :]
  y = y - jnp.mean(y, axis=0, keepdims=True)
  out[...] = y * lax.rsqrt(1e-5 + jnp.sum(y * y) / N)


_tc_post = pl.pallas_call(
    _tc_post_body,
    out_shape=jax.ShapeDtypeStruct((N, D), _F32),
)


def kernel(x0, x1, P0_W, P0_b, P1_W, P1_b, bases0, comp0, root0, bias0,
           bases1, comp1, root1, bias1, edge_index, edge_type):
  src2d = edge_index[0].astype(_I32).reshape(E // D, D)
  typ2d = edge_type.astype(_I32).reshape(E // D, D)
  dst = edge_index[1].astype(_I32)
  rowidx = jnp.arange(N, dtype=_I32)
  zrow = jnp.zeros((ZR, D), _F32)
  zz = jnp.zeros((HPAD,), _F32)

  lat0, hall0, xr0, gidx2d = _tc_pre(x0, x1, P0_W, P0_b, P1_W, P1_b,
                                     bases0, comp0, root0, src2d, typ2d)
  gidx = gidx2d.reshape(E)
  (degp,) = _make_sc_deg()(dst, zz)
  degp = degp.reshape(NW, HPAD // 128, 128)
  dinv = _tc_degsum(degp).reshape(HPAD)[0:N][:, None]
  (aggp0,) = _make_sc_agg()(hall0, gidx, dst, rowidx, zrow)
  aggp0 = aggp0.reshape(NC, N, D)
  lat1, hall1, xr1 = _tc_mid(aggp0, dinv, xr0, bias0, bases1, comp1, root1)
  (aggp1,) = _make_sc_agg()(hall1, gidx, dst, rowidx, zrow)
  aggp1 = aggp1.reshape(NC, N, D)
  out = _tc_post(aggp1, dinv, xr1, bias1)
  return (out, lat0, lat1)


# speedup vs baseline: 27.6932x; 1.0843x over previous
import functools

import jax
import jax.numpy as jnp
from jax import lax
from jax.experimental import pallas as pl
from jax.experimental.pallas import tpu as pltpu
from jax.experimental.pallas import tpu_sc as plsc

N0, N1 = 6000, 4000
N = N0 + N1
E = 320000
D = 128
R = 3

NC, NS = 2, 16
NW = NC * NS
EPT = E // NW
CHUNK = 80
NCHUNK = EPT // CHUNK
ZR = 80
NZB = N // ZR
BPT = (NZB + NS - 1) // NS
HPAD = 10240

_F32 = jnp.float32
_I32 = jnp.int32


@functools.lru_cache(maxsize=None)
def _make_sc_agg(with_deg):
  mesh = plsc.VectorSubcoreMesh(core_axis_name="c", subcore_axis_name="s")
  out_type = [jax.ShapeDtypeStruct((NC * N, D), _F32)]
  if with_deg:
    out_type.append(jax.ShapeDtypeStruct((NW * HPAD,), _F32))
  scratch = [
      pltpu.VMEM((CHUNK,), _I32),
      pltpu.VMEM((CHUNK,), _I32),
      pltpu.VMEM((CHUNK,), _I32),
      pltpu.VMEM((CHUNK,), _I32),
      pltpu.VMEM((ZR,), _I32),
      pltpu.VMEM((CHUNK, D), _F32),
      pltpu.VMEM((CHUNK, D), _F32),
      pltpu.VMEM((ZR, D), _F32),
      pltpu.VMEM_SHARED((N, D), _F32),
      pltpu.SemaphoreType.DMA,
      pltpu.SemaphoreType.DMA,
  ]
  if with_deg:
    scratch.append(pltpu.VMEM((HPAD,), _F32))

  def body(hall, gidx_h, dst_h, rowidx_h, zrow_h, zz_h, *rest):
    if with_deg:
      (agg_out, deg_out, gidx_v0, dst_v0, gidx_v1, dst_v1, ridx_v,
       rows_v0, rows_v1, zrow_v, acc, sem0, sem1, hist_v) = rest
    else:
      (agg_out, gidx_v0, dst_v0, gidx_v1, dst_v1, ridx_v,
       rows_v0, rows_v1, zrow_v, acc, sem0, sem1) = rest
    c = lax.axis_index("c")
    s = lax.axis_index("s")
    wid = c * NS + s
    base = wid * EPT

    pltpu.sync_copy(zrow_h, zrow_v)
    if with_deg:
      pltpu.sync_copy(zz_h, hist_v)

    for k in range(BPT):
      blk = jnp.minimum(s * BPT + k, NZB - 1)
      pltpu.sync_copy(rowidx_h.at[pl.ds(blk * ZR, ZR)], ridx_v)
      pltpu.sync_copy(zrow_v, acc.at[ridx_v])
    plsc.subcore_barrier()

    bufs = ((gidx_v0, dst_v0, rows_v0, sem0),
            (gidx_v1, dst_v1, rows_v1, sem1))

    def start(i, b):
      gv, dv, rv, sm = bufs[b]
      off = base + i * CHUNK
      pltpu.sync_copy(gidx_h.at[pl.ds(off, CHUNK)], gv)
      pltpu.sync_copy(dst_h.at[pl.ds(off, CHUNK)], dv)
      pltpu.async_copy(hall.at[gv], rv, sm)

    ones = jnp.ones((16,), _F32)

    def finish(b):
      gv, dv, rv, sm = bufs[b]
      pltpu.make_async_copy(hall.at[gv], rv, sm).wait()
      pltpu.sync_copy(rv, acc.at[dv], add=True)
      if with_deg:
        for j in range(CHUNK // 16):
          d = dv[pl.ds(j * 16, 16)]
          plsc.addupdate_scatter(hist_v, [d], ones)

    start(0, 0)

    def chunk_pair(t, _):
      g = 2 * t
      start(g + 1, 1)
      finish(0)
      start(g + 2, 0)
      finish(1)
      return 0
    lax.fori_loop(0, (NCHUNK - 1) // 2, chunk_pair, 0)
    finish(0)

    plsc.subcore_barrier()
    for k in range(BPT):
      blk = jnp.minimum(s * BPT + k, NZB - 1)
      pltpu.sync_copy(rowidx_h.at[pl.ds(blk * ZR, ZR)], ridx_v)
      pltpu.async_copy(acc.at[ridx_v], rows_v0, sem0).wait()
      pltpu.sync_copy(rows_v0, agg_out.at[pl.ds(c * N + blk * ZR, ZR)])
    if with_deg:
      pltpu.sync_copy(hist_v, deg_out.at[pl.ds(wid * HPAD, HPAD)])

  return pl.kernel(body, out_type=out_type, mesh=mesh,
                   compiler_params=pltpu.CompilerParams(
                       needs_layout_passes=False),
                   scratch_types=scratch)


def _dot(a, b):
  return jnp.dot(a, b, preferred_element_type=_F32)


def _tc_pre_body(x0, x1, p0w, p0b, p1w, p1b, bases, comp, root, src2d, typ2d,
                 lat0, hall, xr, gidx2d):
  h0 = jnp.maximum(_dot(x0[...], p0w[...]) + p0b[...][None, :], 0.0)
  h1 = jnp.maximum(_dot(x1[...], p1w[...]) + p1b[...][None, :], 0.0)
  lat0[0:N0, :] = h0
  lat0[N0:N, :] = h1
  x = jnp.concatenate([h0, h1], axis=0)
  cm = comp[...]
  b = bases[...]
  for r in range(R):
    w_r = cm[r, 0] * b[0] + cm[r, 1] * b[1]
    hall[r * N:(r + 1) * N, :] = _dot(x, w_r)
  xr[...] = _dot(x, root[...])
  gidx2d[...] = typ2d[...] * N + src2d[...]


_tc_pre = pl.pallas_call(
    _tc_pre_body,
    out_shape=[
        jax.ShapeDtypeStruct((N, D), _F32),
        jax.ShapeDtypeStruct((R * N, D), _F32),
        jax.ShapeDtypeStruct((N, D), _F32),
        jax.ShapeDtypeStruct((E // D, D), _I32),
    ],
)


def _tc_degsum_body(degp, dinv):
  dinv[...] = 1.0 / jnp.maximum(jnp.sum(degp[...], axis=0), 1.0)


_tc_degsum = pl.pallas_call(
    _tc_degsum_body,
    out_shape=jax.ShapeDtypeStruct((HPAD // 128, 128), _F32),
)


def _tc_mid_body(aggp, dinv, xr0, bias0, bases, comp, root,
                 lat1, hall, xr1):
  a = aggp[...]
  y = (a[0] + a[1]) * dinv[...] + xr0[...] + bias0[...][None, :]
  y = jnp.maximum(y, 0.0)
  y = y - jnp.mean(y, axis=0, keepdims=True)
  y = y * lax.rsqrt(1e-5 + jnp.sum(y * y) / N)
  lat1[...] = y
  cm = comp[...]
  b = bases[...]
  for r in range(R):
    w_r = cm[r, 0] * b[0] + cm[r, 1] * b[1]
    hall[r * N:(r + 1) * N, :] = _dot(y, w_r)
  xr1[...] = _dot(y, root[...])


_tc_mid = pl.pallas_call(
    _tc_mid_body,
    out_shape=[
        jax.ShapeDtypeStruct((N, D), _F32),
        jax.ShapeDtypeStruct((R * N, D), _F32),
        jax.ShapeDtypeStruct((N, D), _F32),
    ],
)


def _tc_post_body(aggp, dinv, xr1, bias1, out):
  a = aggp[...]
  y = (a[0] + a[1]) * dinv[...] + xr1[...] + bias1[...][None, :]
  y = y - jnp.mean(y, axis=0, keepdims=True)
  out[...] = y * lax.rsqrt(1e-5 + jnp.sum(y * y) / N)


_tc_post = pl.pallas_call(
    _tc_post_body,
    out_shape=jax.ShapeDtypeStruct((N, D), _F32),
)


def kernel(x0, x1, P0_W, P0_b, P1_W, P1_b, bases0, comp0, root0, bias0,
           bases1, comp1, root1, bias1, edge_index, edge_type):
  src2d = edge_index[0].astype(_I32).reshape(E // D, D)
  typ2d = edge_type.astype(_I32).reshape(E // D, D)
  dst = edge_index[1].astype(_I32)
  rowidx = jnp.arange(N, dtype=_I32)
  zrow = jnp.zeros((ZR, D), _F32)
  zz = jnp.zeros((HPAD,), _F32)

  lat0, hall0, xr0, gidx2d = _tc_pre(x0, x1, P0_W, P0_b, P1_W, P1_b,
                                     bases0, comp0, root0, src2d, typ2d)
  gidx = gidx2d.reshape(E)
  aggp0, degp = _make_sc_agg(True)(hall0, gidx, dst, rowidx, zrow, zz)
  degp = degp.reshape(NW, HPAD // 128, 128)
  dinv = _tc_degsum(degp).reshape(HPAD)[0:N][:, None]
  aggp0 = aggp0.reshape(NC, N, D)
  lat1, hall1, xr1 = _tc_mid(aggp0, dinv, xr0, bias0, bases1, comp1, root1)
  (aggp1,) = _make_sc_agg(False)(hall1, gidx, dst, rowidx, zrow, zz)
  aggp1 = aggp1.reshape(NC, N, D)
  out = _tc_post(aggp1, dinv, xr1, bias1)
  return (out, lat0, lat1)
